# baseline (speedup 1.0000x reference)
import functools

import jax
import jax.numpy as jnp
from jax import lax
from jax.experimental import pallas as pl
from jax.experimental.pallas import tpu as pltpu
from jax.experimental.pallas import tpu_sc as plsc

MAX_LEN = 8192
HALF = MAX_LEN // 2
D_MODEL = 768
BATCH = 4
NUM_CORES = 2
NUM_SUBCORES = 16
NW = NUM_CORES * NUM_SUBCORES
ROWS_PER_W = HALF // NW
ROWS_PER_SC = HALF // NUM_CORES

_mesh = plsc.VectorSubcoreMesh(core_axis_name="c", subcore_axis_name="s")


@functools.partial(
    pl.kernel,
    mesh=_mesh,
    out_type=jax.ShapeDtypeStruct((BATCH * MAX_LEN, D_MODEL), jnp.float32),
    scratch_types=[
        pltpu.VMEM((ROWS_PER_W, D_MODEL), jnp.float32),
        pltpu.VMEM((ROWS_PER_W,), jnp.int32),
        pltpu.VMEM((ROWS_PER_W,), jnp.int32),
        pltpu.VMEM((ROWS_PER_W,), jnp.int32),
        pltpu.VMEM((ROWS_PER_W,), jnp.int32),
        pltpu.VMEM((16, D_MODEL), jnp.float32),
        pltpu.VMEM((16,), jnp.int32),
        pltpu.VMEM((16,), jnp.int32),
        pltpu.SemaphoreType.DMA,
        pltpu.SemaphoreType.DMA,
        pltpu.SemaphoreType.DMA,
    ],
)
def _rel_pos_emb(table_hbm, out_hbm, rows_v, i0, i1, i2, i3,
                 spec_v, gidx, oidx, sem, psem, wsem):
    cid = lax.axis_index("c")
    sid = lax.axis_index("s")
    wid = sid * NUM_CORES + cid
    rbase = wid * ROWS_PER_W
    shbase = cid * ROWS_PER_SC

    @pl.when(sid == 0)
    def _forward():
        for b in range(BATCH):
            pltpu.async_copy(
                table_hbm.at[pl.ds(shbase, ROWS_PER_SC)],
                out_hbm.at[pl.ds(b * MAX_LEN + HALF + shbase, ROWS_PER_SC)],
                wsem)

    pltpu.sync_copy(table_hbm.at[pl.ds(rbase, ROWS_PER_W)], rows_v)

    ridx = [i0, i1, i2, i3]
    for b in range(BATCH):
        for t in range(ROWS_PER_W // 16):
            head = b * MAX_LEN + HALF - rbase - t * 16
            ridx[b][pl.ds(t * 16, 16)] = head - lax.iota(jnp.int32, 16)

    copies = []
    for b in range(BATCH):
        copies.append(pltpu.async_copy(rows_v, out_hbm.at[ridx[b]], sem))

    @pl.when(wid < BATCH)
    def _patch():
        gidx[...] = HALF - lax.iota(jnp.int32, 16)
        oidx[...] = wid * MAX_LEN + lax.iota(jnp.int32, 16)
        pltpu.async_copy(table_hbm.at[gidx], spec_v, psem).wait()
        pltpu.async_copy(spec_v, out_hbm.at[oidx], psem).wait()

    @pl.when(sid == 0)
    def _forward_drain():
        for b in range(BATCH):
            pltpu.make_async_copy(
                table_hbm.at[pl.ds(shbase, ROWS_PER_SC)],
                out_hbm.at[pl.ds(b * MAX_LEN + HALF + shbase, ROWS_PER_SC)],
                wsem).wait()

    for c in copies:
        c.wait()


def kernel(x, table):
    del x
    out = _rel_pos_emb(table)
    return out.reshape(BATCH, MAX_LEN, D_MODEL)


# speedup vs baseline: 19.9689x; 19.9689x over previous
import functools

import jax
import jax.numpy as jnp
from jax import lax
from jax.experimental import pallas as pl
from jax.experimental.pallas import tpu as pltpu
from jax.experimental.pallas import tpu_sc as plsc

MAX_LEN = 8192
HALF = MAX_LEN // 2
D_MODEL = 768
BATCH = 4
NUM_CORES = 2
NUM_SUBCORES = 16
NW = NUM_CORES * NUM_SUBCORES
ROWS_PER_W = HALF // NW
ROWS_PER_SC = HALF // NUM_CORES
SH_CHUNK = 128
N_SH_CHUNK = ROWS_PER_SC // SH_CHUNK

_mesh = plsc.VectorSubcoreMesh(core_axis_name="c", subcore_axis_name="s")


@functools.partial(
    pl.kernel,
    mesh=_mesh,
    out_type=jax.ShapeDtypeStruct((BATCH * MAX_LEN, D_MODEL), jnp.float32),
    scratch_types=[
        pltpu.VMEM((ROWS_PER_W, D_MODEL), jnp.float32),
        pltpu.VMEM((ROWS_PER_W,), jnp.int32),
        pltpu.VMEM((ROWS_PER_W,), jnp.int32),
        pltpu.VMEM((ROWS_PER_W,), jnp.int32),
        pltpu.VMEM((ROWS_PER_W,), jnp.int32),
        pltpu.VMEM((16, D_MODEL), jnp.float32),
        pltpu.VMEM((16,), jnp.int32),
        pltpu.VMEM((16,), jnp.int32),
        pltpu.VMEM_SHARED((SH_CHUNK, D_MODEL), jnp.float32),
        pltpu.VMEM_SHARED((SH_CHUNK, D_MODEL), jnp.float32),
        pltpu.SemaphoreType.DMA,
        pltpu.SemaphoreType.DMA,
        pltpu.SemaphoreType.DMA,
        pltpu.SemaphoreType.DMA,
        pltpu.SemaphoreType.DMA,
    ],
)
def _rel_pos_emb(table_hbm, out_hbm, rows_v, i0, i1, i2, i3,
                 spec_v, gidx, oidx, sh_a, sh_b,
                 sem, psem, ssem, wsem_a, wsem_b):
    cid = lax.axis_index("c")
    sid = lax.axis_index("s")
    wid = sid * NUM_CORES + cid
    rbase = wid * ROWS_PER_W
    score_base = cid * ROWS_PER_SC

    sh = [sh_a, sh_b]
    wsems = [wsem_a, wsem_b]

    def stage_desc(c):
        return pltpu.make_async_copy(
            table_hbm.at[pl.ds(score_base + c * SH_CHUNK, SH_CHUNK)],
            sh[c % 2], ssem)

    def fwd_desc(c, b):
        return pltpu.make_async_copy(
            sh[c % 2],
            out_hbm.at[pl.ds(
                b * MAX_LEN + HALF + score_base + c * SH_CHUNK, SH_CHUNK)],
            wsems[c % 2])

    @pl.when(sid == 0)
    def _stage0():
        stage_desc(0).start()

    pltpu.sync_copy(table_hbm.at[pl.ds(rbase, ROWS_PER_W)], rows_v)

    ridx = [i0, i1, i2, i3]
    for b in range(BATCH):
        for t in range(ROWS_PER_W // 16):
            head = b * MAX_LEN + HALF - rbase - t * 16
            ridx[b][pl.ds(t * 16, 16)] = head - lax.iota(jnp.int32, 16)

    copies = []
    for b in range(BATCH):
        copies.append(pltpu.async_copy(rows_v, out_hbm.at[ridx[b]], sem))

    @pl.when(wid < BATCH)
    def _patch():
        gidx[...] = HALF - lax.iota(jnp.int32, 16)
        oidx[...] = wid * MAX_LEN + lax.iota(jnp.int32, 16)
        pltpu.async_copy(table_hbm.at[gidx], spec_v, psem).wait()
        pltpu.async_copy(spec_v, out_hbm.at[oidx], psem).wait()

    @pl.when(sid == 0)
    def _forward():
        for c in range(N_SH_CHUNK):
            stage_desc(c).wait()
            for b in range(BATCH):
                fwd_desc(c, b).start()
            if c + 1 < N_SH_CHUNK:
                if c >= 1:
                    for b in range(BATCH):
                        fwd_desc(c - 1, b).wait()
                stage_desc(c + 1).start()
        for b in range(BATCH):
            fwd_desc(N_SH_CHUNK - 1, b).wait()

    for c in copies:
        c.wait()


def kernel(x, table):
    del x
    out = _rel_pos_emb(table)
    return out.reshape(BATCH, MAX_LEN, D_MODEL)


# speedup vs baseline: 22.0519x; 1.1043x over previous
import functools

import jax
import jax.numpy as jnp
from jax import lax
from jax.experimental import pallas as pl
from jax.experimental.pallas import tpu as pltpu
from jax.experimental.pallas import tpu_sc as plsc

MAX_LEN = 8192
HALF = MAX_LEN // 2
D_MODEL = 768
BATCH = 4
NUM_CORES = 2
NUM_SUBCORES = 16
NW = NUM_CORES * NUM_SUBCORES
ROWS_PER_W = HALF // NW
ROWS_PER_SC = HALF // NUM_CORES
NMGR = 4
MGR_ROWS = ROWS_PER_SC // NMGR
SH_CHUNK = 32
N_SH_CHUNK = MGR_ROWS // SH_CHUNK

_mesh = plsc.VectorSubcoreMesh(core_axis_name="c", subcore_axis_name="s")


@functools.partial(
    pl.kernel,
    mesh=_mesh,
    out_type=jax.ShapeDtypeStruct((BATCH * MAX_LEN, D_MODEL), jnp.float32),
    scratch_types=[
        pltpu.VMEM((ROWS_PER_W, D_MODEL), jnp.float32),
        pltpu.VMEM((ROWS_PER_W,), jnp.int32),
        pltpu.VMEM((ROWS_PER_W,), jnp.int32),
        pltpu.VMEM((ROWS_PER_W,), jnp.int32),
        pltpu.VMEM((ROWS_PER_W,), jnp.int32),
        pltpu.VMEM((16, D_MODEL), jnp.float32),
        pltpu.VMEM((16,), jnp.int32),
        pltpu.VMEM((16,), jnp.int32),
        pltpu.VMEM_SHARED((NMGR * 2 * SH_CHUNK, D_MODEL), jnp.float32),
        pltpu.SemaphoreType.DMA,
        pltpu.SemaphoreType.DMA,
        pltpu.SemaphoreType.DMA,
        pltpu.SemaphoreType.DMA,
        pltpu.SemaphoreType.DMA,
    ],
)
def _rel_pos_emb(table_hbm, out_hbm, rows_v, i0, i1, i2, i3,
                 spec_v, gidx, oidx, sh_v,
                 sem, psem, ssem, wsem_a, wsem_b):
    cid = lax.axis_index("c")
    sid = lax.axis_index("s")
    wid = sid * NUM_CORES + cid
    rbase = wid * ROWS_PER_W
    mgr_base = cid * ROWS_PER_SC + sid * MGR_ROWS

    wsems = [wsem_a, wsem_b]

    def sh_slot(c):
        return sh_v.at[pl.ds((sid * 2 + c % 2) * SH_CHUNK, SH_CHUNK)]

    def stage_desc(c):
        return pltpu.make_async_copy(
            table_hbm.at[pl.ds(mgr_base + c * SH_CHUNK, SH_CHUNK)],
            sh_slot(c), ssem)

    def fwd_desc(c, b):
        return pltpu.make_async_copy(
            sh_slot(c),
            out_hbm.at[pl.ds(
                b * MAX_LEN + HALF + mgr_base + c * SH_CHUNK, SH_CHUNK)],
            wsems[c % 2])

    @pl.when(sid < NMGR)
    def _stage0():
        stage_desc(0).start()

    pltpu.sync_copy(table_hbm.at[pl.ds(rbase, ROWS_PER_W)], rows_v)

    ridx = [i0, i1, i2, i3]
    for b in range(BATCH):
        for t in range(ROWS_PER_W // 16):
            head = b * MAX_LEN + HALF - rbase - t * 16
            ridx[b][pl.ds(t * 16, 16)] = head - lax.iota(jnp.int32, 16)

    copies = []
    for b in range(BATCH):
        copies.append(pltpu.async_copy(rows_v, out_hbm.at[ridx[b]], sem))

    @pl.when(wid < BATCH)
    def _patch():
        gidx[...] = HALF - lax.iota(jnp.int32, 16)
        oidx[...] = wid * MAX_LEN + lax.iota(jnp.int32, 16)
        pltpu.async_copy(table_hbm.at[gidx], spec_v, psem).wait()
        pltpu.async_copy(spec_v, out_hbm.at[oidx], psem).wait()

    @pl.when(sid < NMGR)
    def _forward():
        for c in range(N_SH_CHUNK):
            stage_desc(c).wait()
            for b in range(BATCH):
                fwd_desc(c, b).start()
            if c + 1 < N_SH_CHUNK:
                if c >= 1:
                    for b in range(BATCH):
                        fwd_desc(c - 1, b).wait()
                stage_desc(c + 1).start()
        for b in range(BATCH):
            fwd_desc(N_SH_CHUNK - 2, b).wait()
        for b in range(BATCH):
            fwd_desc(N_SH_CHUNK - 1, b).wait()

    for c in copies:
        c.wait()


def kernel(x, table):
    del x
    out = _rel_pos_emb(table)
    return out.reshape(BATCH, MAX_LEN, D_MODEL)


# speedup vs baseline: 25.6945x; 1.1652x over previous
import functools

import jax
import jax.numpy as jnp
from jax import lax
from jax.experimental import pallas as pl
from jax.experimental.pallas import tpu as pltpu
from jax.experimental.pallas import tpu_sc as plsc

MAX_LEN = 8192
HALF = MAX_LEN // 2
D_MODEL = 768
BATCH = 4
NUM_CORES = 2
NUM_SUBCORES = 16
NW = NUM_CORES * NUM_SUBCORES
ROWS_PER_W = HALF // NW
ROWS_PER_SC = HALF // NUM_CORES
FWD_TILE = 80
SH_CHUNK = ROWS_PER_W - FWD_TILE
NMGR = 4
BLK_PER_MGR = NUM_SUBCORES // NMGR

_mesh = plsc.VectorSubcoreMesh(core_axis_name="c", subcore_axis_name="s")


@functools.partial(
    pl.kernel,
    mesh=_mesh,
    out_type=jax.ShapeDtypeStruct((BATCH * MAX_LEN, D_MODEL), jnp.float32),
    scratch_types=[
        pltpu.VMEM((ROWS_PER_W, D_MODEL), jnp.float32),
        pltpu.VMEM((ROWS_PER_W,), jnp.int32),
        pltpu.VMEM((ROWS_PER_W,), jnp.int32),
        pltpu.VMEM((ROWS_PER_W,), jnp.int32),
        pltpu.VMEM((ROWS_PER_W,), jnp.int32),
        pltpu.VMEM((16, D_MODEL), jnp.float32),
        pltpu.VMEM((16,), jnp.int32),
        pltpu.VMEM((16,), jnp.int32),
        pltpu.VMEM_SHARED((NMGR * 2 * SH_CHUNK, D_MODEL), jnp.float32),
        pltpu.SemaphoreType.DMA,
        pltpu.SemaphoreType.DMA,
        pltpu.SemaphoreType.DMA,
        pltpu.SemaphoreType.DMA,
        pltpu.SemaphoreType.DMA,
    ],
)
def _rel_pos_emb(table_hbm, out_hbm, rows_v, i0, i1, i2, i3,
                 spec_v, gidx, oidx, sh_v,
                 sem, psem, ssem, wsem_a, wsem_b):
    cid = lax.axis_index("c")
    sid = lax.axis_index("s")
    wid = sid * NUM_CORES + cid
    rbase = wid * ROWS_PER_W
    wsems = [wsem_a, wsem_b]

    def blk_row(c):
        return (cid * NUM_SUBCORES + sid * BLK_PER_MGR + c) * ROWS_PER_W \
            + FWD_TILE

    def sh_slot(c):
        return sh_v.at[pl.ds((sid * 2 + c % 2) * SH_CHUNK, SH_CHUNK)]

    def stage_desc(c):
        return pltpu.make_async_copy(
            table_hbm.at[pl.ds(blk_row(c), SH_CHUNK)], sh_slot(c), ssem)

    def fwd_desc(c, b):
        return pltpu.make_async_copy(
            sh_slot(c),
            out_hbm.at[pl.ds(b * MAX_LEN + HALF + blk_row(c), SH_CHUNK)],
            wsems[c % 2])

    @pl.when(sid < NMGR)
    def _stage0():
        stage_desc(0).start()

    pltpu.sync_copy(table_hbm.at[pl.ds(rbase, ROWS_PER_W)], rows_v)

    ridx = [i0, i1, i2, i3]
    for b in range(BATCH):
        for t in range(ROWS_PER_W // 16):
            head = b * MAX_LEN + HALF - rbase - t * 16
            ridx[b][pl.ds(t * 16, 16)] = head - lax.iota(jnp.int32, 16)

    copies = []
    for b in range(BATCH):
        copies.append(pltpu.async_copy(rows_v, out_hbm.at[ridx[b]], sem))
        copies.append(pltpu.async_copy(
            rows_v.at[pl.ds(0, FWD_TILE)],
            out_hbm.at[pl.ds(b * MAX_LEN + HALF + rbase, FWD_TILE)],
            sem))

    @pl.when(wid < BATCH)
    def _patch():
        gidx[...] = HALF - lax.iota(jnp.int32, 16)
        oidx[...] = wid * MAX_LEN + lax.iota(jnp.int32, 16)
        pltpu.async_copy(table_hbm.at[gidx], spec_v, psem).wait()
        pltpu.async_copy(spec_v, out_hbm.at[oidx], psem).wait()

    @pl.when(sid < NMGR)
    def _forward():
        for c in range(BLK_PER_MGR):
            stage_desc(c).wait()
            for b in range(BATCH):
                fwd_desc(c, b).start()
            if c + 1 < BLK_PER_MGR:
                if c >= 1:
                    for b in range(BATCH):
                        fwd_desc(c - 1, b).wait()
                stage_desc(c + 1).start()
        for b in range(BATCH):
            fwd_desc(BLK_PER_MGR - 2, b).wait()
        for b in range(BATCH):
            fwd_desc(BLK_PER_MGR - 1, b).wait()

    for c in copies:
        c.wait()


def kernel(x, table):
    del x
    out = _rel_pos_emb(table)
    return out.reshape(BATCH, MAX_LEN, D_MODEL)


# speedup vs baseline: 27.7418x; 1.0797x over previous
import functools

import jax
import jax.numpy as jnp
from jax import lax
from jax.experimental import pallas as pl
from jax.experimental.pallas import tpu as pltpu
from jax.experimental.pallas import tpu_sc as plsc

MAX_LEN = 8192
HALF = MAX_LEN // 2
D_MODEL = 768
BATCH = 4
NUM_CORES = 2
NUM_SUBCORES = 16
NW = NUM_CORES * NUM_SUBCORES
ROWS_PER_W = HALF // NW
CHUNK = ROWS_PER_W // 2

_mesh = plsc.VectorSubcoreMesh(core_axis_name="c", subcore_axis_name="s")


@functools.partial(
    pl.kernel,
    mesh=_mesh,
    out_type=jax.ShapeDtypeStruct((BATCH * MAX_LEN, D_MODEL), jnp.float32),
    scratch_types=[
        pltpu.VMEM((CHUNK, D_MODEL), jnp.float32),
        pltpu.VMEM((CHUNK, D_MODEL), jnp.float32),
        pltpu.VMEM((CHUNK,), jnp.int32),
        pltpu.VMEM((CHUNK,), jnp.int32),
        pltpu.VMEM((CHUNK,), jnp.int32),
        pltpu.VMEM((CHUNK,), jnp.int32),
        pltpu.VMEM((CHUNK,), jnp.int32),
        pltpu.VMEM((CHUNK,), jnp.int32),
        pltpu.VMEM((CHUNK,), jnp.int32),
        pltpu.VMEM((CHUNK,), jnp.int32),
        pltpu.VMEM((16, D_MODEL), jnp.float32),
        pltpu.VMEM((16,), jnp.int32),
        pltpu.VMEM((16,), jnp.int32),
        pltpu.SemaphoreType.DMA,
        pltpu.SemaphoreType.DMA,
        pltpu.SemaphoreType.DMA,
    ],
)
def _rel_pos_emb(table_hbm, out_hbm, rows_a, rows_b,
                 ia0, ia1, ia2, ia3, ib0, ib1, ib2, ib3,
                 spec_v, gidx, oidx, sem, psem, rsem):
    wid = lax.axis_index("s") * NUM_CORES + lax.axis_index("c")
    rbase = wid * ROWS_PER_W

    rows = [rows_a, rows_b]
    ridx = [[ia0, ia1, ia2, ia3], [ib0, ib1, ib2, ib3]]

    def read_desc(c):
        return pltpu.make_async_copy(
            table_hbm.at[pl.ds(rbase + c * CHUNK, CHUNK)], rows[c], rsem)

    read_desc(0).start()
    read_desc(1).start()

    for c in range(2):
        for b in range(BATCH):
            for t in range(CHUNK // 16):
                head = b * MAX_LEN + HALF - rbase - c * CHUNK - t * 16
                ridx[c][b][pl.ds(t * 16, 16)] = head - lax.iota(jnp.int32, 16)

    copies = []
    for c in range(2):
        read_desc(c).wait()
        for b in range(BATCH):
            copies.append(pltpu.async_copy(rows[c], out_hbm.at[ridx[c][b]],
                                           sem))
            copies.append(pltpu.async_copy(
                rows[c],
                out_hbm.at[pl.ds(b * MAX_LEN + HALF + rbase + c * CHUNK,
                                 CHUNK)],
                sem))

    @pl.when(wid < BATCH)
    def _patch():
        gidx[...] = HALF - lax.iota(jnp.int32, 16)
        oidx[...] = wid * MAX_LEN + lax.iota(jnp.int32, 16)
        pltpu.async_copy(table_hbm.at[gidx], spec_v, psem).wait()
        pltpu.async_copy(spec_v, out_hbm.at[oidx], psem).wait()

    for c in copies:
        c.wait()


def kernel(x, table):
    del x
    out = _rel_pos_emb(table)
    return out.reshape(BATCH, MAX_LEN, D_MODEL)
